# Initial kernel scaffold; baseline (speedup 1.0000x reference)
import jax
import jax.numpy as jnp
from jax.experimental import pallas as pl


def kernel(X, edge_index, W0, b0, W1, b1):
    raise NotImplementedError("write your pallas kernel here")


# speedup vs baseline: 14.0282x; 14.0282x over previous
import jax
import jax.numpy as jnp
from jax import lax
from jax.experimental import pallas as pl
from jax.experimental.pallas import tpu as pltpu
from jax.experimental.pallas import tpu_sc as plsc

_N = 10000
_E = 640000
_D_IN = 1433
_D_HID = 140
_D_OUT = 7

_NC = 2
_NS = 16
_NW = _NC * _NS
_NPAD = 10112
_RPT = _NPAD // _NS
_D1 = 144
_D2 = 16
_K = 80
_EPW = _E // _NW


def _deg_body(src_hbm, dst_hbm, deg_out, *scratch):
    G = 25
    idx_s = scratch[:G]
    idx_d = scratch[G:2 * G]
    ones_v, zbuf, sp_o, sp_i, isem, ssem = scratch[2 * G:]
    cid = lax.axis_index("c")
    sid = lax.axis_index("s")
    wid = cid * _NS + sid
    for i in range(_K // 16):
        ones_v[pl.ds(16 * i, 16)] = jnp.ones((16,), jnp.float32)
    for i in range(_RPT // 16):
        zbuf[pl.ds(16 * i, 16)] = jnp.zeros((16,), jnp.float32)
    pltpu.sync_copy(zbuf, sp_o.at[pl.ds(sid * _RPT, _RPT)])
    pltpu.sync_copy(zbuf, sp_i.at[pl.ds(sid * _RPT, _RPT)])
    plsc.subcore_barrier()

    def body(t, carry):
        base = wid * _EPW + t * (G * _K)
        ld = [pltpu.async_copy(src_hbm.at[pl.ds(base + g * _K, _K)],
                               idx_s[g], isem) for g in range(G)]
        ld += [pltpu.async_copy(dst_hbm.at[pl.ds(base + g * _K, _K)],
                                idx_d[g], isem) for g in range(G)]
        for d in ld:
            d.wait()
        sd = [pltpu.async_copy(ones_v, sp_o.at[idx_s[g]], ssem, add=True)
              for g in range(G)]
        sd += [pltpu.async_copy(ones_v, sp_i.at[idx_d[g]], ssem, add=True)
               for g in range(G)]
        for d in sd:
            d.wait()
        return carry

    lax.fori_loop(0, _EPW // (G * _K), body, 0)
    plsc.subcore_barrier()
    off = cid * _NPAD + sid * _RPT
    pltpu.sync_copy(sp_o.at[pl.ds(sid * _RPT, _RPT)],
                    deg_out.at[pl.ds(off, _RPT)])
    pltpu.sync_copy(sp_i.at[pl.ds(sid * _RPT, _RPT)],
                    deg_out.at[pl.ds(2 * _NPAD + off, _RPT)])


_SC_PARAMS = pltpu.CompilerParams(use_tc_tiling_on_sc=False)

_deg_call = pl.kernel(
    _deg_body,
    out_type=jax.ShapeDtypeStruct((4 * _NPAD,), jnp.float32),
    mesh=plsc.VectorSubcoreMesh(core_axis_name="c", subcore_axis_name="s"),
    compiler_params=_SC_PARAMS,
    scratch_types=(
        [pltpu.VMEM((_K,), jnp.int32) for _ in range(50)]
        + [
            pltpu.VMEM((_K,), jnp.float32),
            pltpu.VMEM((_RPT,), jnp.float32),
            pltpu.VMEM_SHARED((_NPAD,), jnp.float32),
            pltpu.VMEM_SHARED((_NPAD,), jnp.float32),
            pltpu.SemaphoreType.DMA,
            pltpu.SemaphoreType.DMA,
        ]
    ),
)


def _make_agg(D, G):
    def body(h_hbm, src_hbm, dst_hbm, agg_out, *scratch):
        idx_s = scratch[:G]
        idx_d = scratch[G:2 * G]
        stage = scratch[2 * G:3 * G]
        z8, sp_agg, isem, gsem, ssem = scratch[3 * G:]
        cid = lax.axis_index("c")
        sid = lax.axis_index("s")
        wid = cid * _NS + sid
        for i in range(8 * D // 16):
            z8[i // (D // 16), pl.ds(16 * (i % (D // 16)), 16)] = (
                jnp.zeros((16,), jnp.float32))
        zd = [pltpu.async_copy(z8, sp_agg.at[pl.ds(sid * _RPT + 8 * r, 8)],
                               gsem) for r in range(_RPT // 8)]
        for d in zd:
            d.wait()
        plsc.subcore_barrier()

        def body_t(t, carry):
            base = wid * _EPW + t * (G * _K)
            ld = [pltpu.async_copy(src_hbm.at[pl.ds(base + g * _K, _K)],
                                   idx_s[g], isem) for g in range(G)]
            ld += [pltpu.async_copy(dst_hbm.at[pl.ds(base + g * _K, _K)],
                                    idx_d[g], isem) for g in range(G)]
            for d in ld:
                d.wait()
            gd = [pltpu.async_copy(h_hbm.at[idx_s[g]], stage[g], gsem)
                  for g in range(G)]
            for d in gd:
                d.wait()
            sd = [pltpu.async_copy(stage[g], sp_agg.at[idx_d[g]], ssem,
                                   add=True) for g in range(G)]
            for d in sd:
                d.wait()
            return carry

        lax.fori_loop(0, _EPW // (G * _K), body_t, 0)
        plsc.subcore_barrier()
        pltpu.sync_copy(sp_agg.at[pl.ds(sid * _RPT, _RPT)],
                        agg_out.at[cid, pl.ds(sid * _RPT, _RPT)])

    return pl.kernel(
        body,
        out_type=jax.ShapeDtypeStruct((_NC, _NPAD, D), jnp.float32),
        mesh=plsc.VectorSubcoreMesh(core_axis_name="c", subcore_axis_name="s"),
        compiler_params=_SC_PARAMS,
        scratch_types=(
            [pltpu.VMEM((_K,), jnp.int32) for _ in range(2 * G)]
            + [pltpu.VMEM((_K, D), jnp.float32) for _ in range(G)]
            + [
                pltpu.VMEM((8, D), jnp.float32),
                pltpu.VMEM_SHARED((_NPAD, D), jnp.float32),
                pltpu.SemaphoreType.DMA,
                pltpu.SemaphoreType.DMA,
                pltpu.SemaphoreType.DMA,
            ]
        ),
    )


_agg_hid = _make_agg(_D1, 2)
_agg_out = _make_agg(_D2, 25)


def _oinv_body(od_ref, o_ref):
    od = od_ref[0, :] + od_ref[1, :]
    o_ref[...] = lax.rsqrt(jnp.maximum(od, 1.0))[:, None]


def _oinv_call(odeg):
    return pl.pallas_call(
        _oinv_body,
        out_shape=jax.ShapeDtypeStruct((_NPAD, 1), jnp.float32),
    )(odeg)


def _mm_body(x_ref, s_ref, w_ref, o_ref):
    xs = x_ref[...] * s_ref[...]
    o_ref[:, :_D_HID] = jnp.dot(xs, w_ref[...],
                                preferred_element_type=jnp.float32)
    o_ref[:, _D_HID:] = jnp.zeros((1000, _D1 - _D_HID), jnp.float32)


def _mm_call(X, oinv, W0):
    return pl.pallas_call(
        _mm_body,
        grid=(10,),
        in_specs=[pl.BlockSpec((1000, _D_IN), lambda i: (i, 0)),
                  pl.BlockSpec((1000, 1), lambda i: (i, 0)),
                  pl.BlockSpec((_D_IN, _D_HID), lambda i: (0, 0))],
        out_specs=pl.BlockSpec((1000, _D1), lambda i: (i, 0)),
        out_shape=jax.ShapeDtypeStruct((_N, _D1), jnp.float32),
    )(X, oinv, W0)


def _l2_body(agg_ref, od_ref, id_ref, b0_ref, w1_ref, h2_ref):
    a = agg_ref[0, :_N, :_D_HID] + agg_ref[1, :_N, :_D_HID]
    iinv = lax.rsqrt(jnp.maximum(id_ref[0, :] + id_ref[1, :], 1.0))[:_N, None]
    oinv = lax.rsqrt(jnp.maximum(od_ref[0, :] + od_ref[1, :], 1.0))[:_N, None]
    h1 = jnp.maximum(a * iinv + b0_ref[...][None, :], 0.0)
    m2 = jnp.dot(h1 * oinv, w1_ref[...], preferred_element_type=jnp.float32)
    h2_ref[:, :_D_OUT] = m2
    h2_ref[:, _D_OUT:] = jnp.zeros((_N, _D2 - _D_OUT), jnp.float32)


def _l2_call(agg1, odeg, ideg, b0, W1):
    return pl.pallas_call(
        _l2_body,
        out_shape=jax.ShapeDtypeStruct((_N, _D2), jnp.float32),
    )(agg1, odeg, ideg, b0, W1)


def _fin_body(agg_ref, id_ref, b1_ref, o_ref):
    a = agg_ref[0, :_N, :_D_OUT] + agg_ref[1, :_N, :_D_OUT]
    iinv = lax.rsqrt(jnp.maximum(id_ref[0, :] + id_ref[1, :], 1.0))[:_N, None]
    o_ref[...] = jnp.maximum(a * iinv + b1_ref[...][None, :], 0.0)


def _fin_call(agg2, ideg, b1):
    return pl.pallas_call(
        _fin_body,
        out_shape=jax.ShapeDtypeStruct((_N, _D_OUT), jnp.float32),
    )(agg2, ideg, b1)


def kernel(X, edge_index, W0, b0, W1, b1):
    src = edge_index[0]
    dst = edge_index[1]
    deg = _deg_call(src, dst)
    odeg = deg[:2 * _NPAD].reshape(2, _NPAD)
    ideg = deg[2 * _NPAD:].reshape(2, _NPAD)
    h1p = _mm_call(X, _oinv_call(odeg), W0)
    agg1 = _agg_hid(h1p, src, dst)
    h2p = _l2_call(agg1, odeg, ideg, b0, W1)
    agg2 = _agg_out(h2p, src, dst)
    return _fin_call(agg2, ideg, b1)


# speedup vs baseline: 16.1281x; 1.1497x over previous
import jax
import jax.numpy as jnp
from jax import lax
from jax.experimental import pallas as pl
from jax.experimental.pallas import tpu as pltpu
from jax.experimental.pallas import tpu_sc as plsc

_N = 10000
_E = 640000
_D_IN = 1433
_D_HID = 140
_D_OUT = 7

_NC = 2
_NS = 16
_NW = _NC * _NS
_NPAD = 10240
_RPT = _NPAD // _NS
_D1 = 144
_D2 = 16
_K = 80
_EPW = _E // _NW


def _deg_body(src_hbm, dst_hbm, deg_out, *scratch):
    G = 25
    idx_s = scratch[:G]
    idx_d = scratch[G:2 * G]
    ones_v, zbuf, sp_o, sp_i, isem, ssem = scratch[2 * G:]
    cid = lax.axis_index("c")
    sid = lax.axis_index("s")
    wid = cid * _NS + sid
    for i in range(_K // 16):
        ones_v[pl.ds(16 * i, 16)] = jnp.ones((16,), jnp.float32)
    for i in range(_RPT // 16):
        zbuf[pl.ds(16 * i, 16)] = jnp.zeros((16,), jnp.float32)
    pltpu.sync_copy(zbuf, sp_o.at[pl.ds(sid * _RPT, _RPT)])
    pltpu.sync_copy(zbuf, sp_i.at[pl.ds(sid * _RPT, _RPT)])
    plsc.subcore_barrier()

    def body(t, carry):
        base = wid * _EPW + t * (G * _K)
        ld = [pltpu.async_copy(src_hbm.at[pl.ds(base + g * _K, _K)],
                               idx_s[g], isem) for g in range(G)]
        ld += [pltpu.async_copy(dst_hbm.at[pl.ds(base + g * _K, _K)],
                                idx_d[g], isem) for g in range(G)]
        for d in ld:
            d.wait()
        sd = [pltpu.async_copy(ones_v, sp_o.at[idx_s[g]], ssem, add=True)
              for g in range(G)]
        sd += [pltpu.async_copy(ones_v, sp_i.at[idx_d[g]], ssem, add=True)
               for g in range(G)]
        for d in sd:
            d.wait()
        return carry

    lax.fori_loop(0, _EPW // (G * _K), body, 0)
    plsc.subcore_barrier()
    off = cid * _NPAD + sid * _RPT
    pltpu.sync_copy(sp_o.at[pl.ds(sid * _RPT, _RPT)],
                    deg_out.at[pl.ds(off, _RPT)])
    pltpu.sync_copy(sp_i.at[pl.ds(sid * _RPT, _RPT)],
                    deg_out.at[pl.ds(2 * _NPAD + off, _RPT)])


_SC_PARAMS = pltpu.CompilerParams(use_tc_tiling_on_sc=False)

_deg_call = pl.kernel(
    _deg_body,
    out_type=jax.ShapeDtypeStruct((4 * _NPAD,), jnp.float32),
    mesh=plsc.VectorSubcoreMesh(core_axis_name="c", subcore_axis_name="s"),
    compiler_params=_SC_PARAMS,
    scratch_types=(
        [pltpu.VMEM((_K,), jnp.int32) for _ in range(50)]
        + [
            pltpu.VMEM((_K,), jnp.float32),
            pltpu.VMEM((_RPT,), jnp.float32),
            pltpu.VMEM_SHARED((_NPAD,), jnp.float32),
            pltpu.VMEM_SHARED((_NPAD,), jnp.float32),
            pltpu.SemaphoreType.DMA,
            pltpu.SemaphoreType.DMA,
        ]
    ),
)


def _agg1_body(h_hbm, src_hbm, dst_hbm, agg_out, *scratch):
    D = _D1
    NCH = _EPW // _K
    idx_s = scratch[0:4]
    idx_d = scratch[4:8]
    stage = scratch[8:10]
    z8, sp_agg, isem, gsem, ssem = scratch[10:]
    cid = lax.axis_index("c")
    sid = lax.axis_index("s")
    wid = cid * _NS + sid
    base = wid * _EPW
    for i in range(8 * D // 16):
        z8[i // (D // 16), pl.ds(16 * (i % (D // 16)), 16)] = (
            jnp.zeros((16,), jnp.float32))
    zd = [pltpu.async_copy(z8, sp_agg.at[pl.ds(sid * _RPT + 8 * r, 8)],
                           gsem) for r in range(_RPT // 8)]
    for d in zd:
        d.wait()
    plsc.subcore_barrier()

    def fire_idx(c, slot):
        pltpu.async_copy(src_hbm.at[pl.ds(base + c * _K, _K)],
                         idx_s[slot], isem)
        pltpu.async_copy(dst_hbm.at[pl.ds(base + c * _K, _K)],
                         idx_d[slot], isem)

    def wait_idx(slot):
        pltpu.make_async_copy(src_hbm.at[pl.ds(base, _K)],
                              idx_s[slot], isem).wait()
        pltpu.make_async_copy(dst_hbm.at[pl.ds(base, _K)],
                              idx_d[slot], isem).wait()

    def wait_scatter(slot2, slot4):
        pltpu.make_async_copy(stage[slot2], sp_agg.at[idx_d[slot4]],
                              ssem).wait()

    for c in range(4):
        fire_idx(c, c)
    for c in range(2):
        wait_idx(c)
        pltpu.async_copy(h_hbm.at[idx_s[c]], stage[c], gsem).wait()
        pltpu.async_copy(stage[c], sp_agg.at[idx_d[c]], ssem, add=True)

    def body_t(t, carry):
        for j in range(4):
            sl4 = (j + 2) % 4
            sl2 = j % 2
            c = 4 * t + j + 2
            wait_idx(sl4)
            wait_scatter(sl2, j)
            g = pltpu.async_copy(h_hbm.at[idx_s[sl4]], stage[sl2], gsem)

            @pl.when(c + 2 <= NCH - 1)
            def _():
                fire_idx(c + 2, j)

            g.wait()
            pltpu.async_copy(stage[sl2], sp_agg.at[idx_d[sl4]], ssem,
                             add=True)
        return carry

    lax.fori_loop(0, (NCH - 2) // 4, body_t, 0)
    wait_scatter(0, 0)
    wait_scatter(1, 1)
    plsc.subcore_barrier()
    pltpu.sync_copy(sp_agg.at[pl.ds(sid * _RPT, _RPT)],
                    agg_out.at[cid, pl.ds(sid * _RPT, _RPT)])


_agg_hid = pl.kernel(
    _agg1_body,
    out_type=jax.ShapeDtypeStruct((_NC, _NPAD, _D1), jnp.float32),
    mesh=plsc.VectorSubcoreMesh(core_axis_name="c", subcore_axis_name="s"),
    compiler_params=_SC_PARAMS,
    scratch_types=(
        [pltpu.VMEM((_K,), jnp.int32) for _ in range(8)]
        + [pltpu.VMEM((_K, _D1), jnp.float32) for _ in range(2)]
        + [
            pltpu.VMEM((8, _D1), jnp.float32),
            pltpu.VMEM_SHARED((_NPAD, _D1), jnp.float32),
            pltpu.SemaphoreType.DMA,
            pltpu.SemaphoreType.DMA,
            pltpu.SemaphoreType.DMA,
        ]
    ),
)


def _make_agg(D, G):
    def body(h_hbm, src_hbm, dst_hbm, agg_out, *scratch):
        idx_s = scratch[:G]
        idx_d = scratch[G:2 * G]
        stage = scratch[2 * G:3 * G]
        z8, sp_agg, isem, gsem, ssem = scratch[3 * G:]
        cid = lax.axis_index("c")
        sid = lax.axis_index("s")
        wid = cid * _NS + sid
        for i in range(8 * D // 16):
            z8[i // (D // 16), pl.ds(16 * (i % (D // 16)), 16)] = (
                jnp.zeros((16,), jnp.float32))
        zd = [pltpu.async_copy(z8, sp_agg.at[pl.ds(sid * _RPT + 8 * r, 8)],
                               gsem) for r in range(_RPT // 8)]
        for d in zd:
            d.wait()
        plsc.subcore_barrier()

        def body_t(t, carry):
            base = wid * _EPW + t * (G * _K)
            ld = [pltpu.async_copy(src_hbm.at[pl.ds(base + g * _K, _K)],
                                   idx_s[g], isem) for g in range(G)]
            ld += [pltpu.async_copy(dst_hbm.at[pl.ds(base + g * _K, _K)],
                                    idx_d[g], isem) for g in range(G)]
            for d in ld:
                d.wait()
            gd = [pltpu.async_copy(h_hbm.at[idx_s[g]], stage[g], gsem)
                  for g in range(G)]
            for d in gd:
                d.wait()
            sd = [pltpu.async_copy(stage[g], sp_agg.at[idx_d[g]], ssem,
                                   add=True) for g in range(G)]
            for d in sd:
                d.wait()
            return carry

        lax.fori_loop(0, _EPW // (G * _K), body_t, 0)
        plsc.subcore_barrier()
        pltpu.sync_copy(sp_agg.at[pl.ds(sid * _RPT, _RPT)],
                        agg_out.at[cid, pl.ds(sid * _RPT, _RPT)])

    return pl.kernel(
        body,
        out_type=jax.ShapeDtypeStruct((_NC, _NPAD, D), jnp.float32),
        mesh=plsc.VectorSubcoreMesh(core_axis_name="c", subcore_axis_name="s"),
        compiler_params=_SC_PARAMS,
        scratch_types=(
            [pltpu.VMEM((_K,), jnp.int32) for _ in range(2 * G)]
            + [pltpu.VMEM((_K, D), jnp.float32) for _ in range(G)]
            + [
                pltpu.VMEM((8, D), jnp.float32),
                pltpu.VMEM_SHARED((_NPAD, D), jnp.float32),
                pltpu.SemaphoreType.DMA,
                pltpu.SemaphoreType.DMA,
                pltpu.SemaphoreType.DMA,
            ]
        ),
    )


_agg_out = _make_agg(_D2, 25)


def _oinv_body(od_ref, o_ref):
    od = od_ref[0, :] + od_ref[1, :]
    o_ref[...] = lax.rsqrt(jnp.maximum(od, 1.0))[:, None]


def _oinv_call(odeg):
    return pl.pallas_call(
        _oinv_body,
        out_shape=jax.ShapeDtypeStruct((_NPAD, 1), jnp.float32),
    )(odeg)


def _mm_body(x_ref, s_ref, w_ref, o_ref):
    xs = x_ref[...] * s_ref[...]
    o_ref[:, :_D_HID] = jnp.dot(xs, w_ref[...],
                                preferred_element_type=jnp.float32)
    o_ref[:, _D_HID:] = jnp.zeros((1000, _D1 - _D_HID), jnp.float32)


def _mm_call(X, oinv, W0):
    return pl.pallas_call(
        _mm_body,
        grid=(10,),
        in_specs=[pl.BlockSpec((1000, _D_IN), lambda i: (i, 0)),
                  pl.BlockSpec((1000, 1), lambda i: (i, 0)),
                  pl.BlockSpec((_D_IN, _D_HID), lambda i: (0, 0))],
        out_specs=pl.BlockSpec((1000, _D1), lambda i: (i, 0)),
        out_shape=jax.ShapeDtypeStruct((_N, _D1), jnp.float32),
    )(X, oinv, W0)


def _l2_body(agg_ref, od_ref, id_ref, b0_ref, w1_ref, h2_ref):
    a = agg_ref[0, :_N, :_D_HID] + agg_ref[1, :_N, :_D_HID]
    iinv = lax.rsqrt(jnp.maximum(id_ref[0, :] + id_ref[1, :], 1.0))[:_N, None]
    oinv = lax.rsqrt(jnp.maximum(od_ref[0, :] + od_ref[1, :], 1.0))[:_N, None]
    h1 = jnp.maximum(a * iinv + b0_ref[...][None, :], 0.0)
    m2 = jnp.dot(h1 * oinv, w1_ref[...], preferred_element_type=jnp.float32)
    h2_ref[:, :_D_OUT] = m2
    h2_ref[:, _D_OUT:] = jnp.zeros((_N, _D2 - _D_OUT), jnp.float32)


def _l2_call(agg1, odeg, ideg, b0, W1):
    return pl.pallas_call(
        _l2_body,
        out_shape=jax.ShapeDtypeStruct((_N, _D2), jnp.float32),
    )(agg1, odeg, ideg, b0, W1)


def _fin_body(agg_ref, id_ref, b1_ref, o_ref):
    a = agg_ref[0, :_N, :_D_OUT] + agg_ref[1, :_N, :_D_OUT]
    iinv = lax.rsqrt(jnp.maximum(id_ref[0, :] + id_ref[1, :], 1.0))[:_N, None]
    o_ref[...] = jnp.maximum(a * iinv + b1_ref[...][None, :], 0.0)


def _fin_call(agg2, ideg, b1):
    return pl.pallas_call(
        _fin_body,
        out_shape=jax.ShapeDtypeStruct((_N, _D_OUT), jnp.float32),
    )(agg2, ideg, b1)


def kernel(X, edge_index, W0, b0, W1, b1):
    src = edge_index[0]
    dst = edge_index[1]
    deg = _deg_call(src, dst)
    odeg = deg[:2 * _NPAD].reshape(2, _NPAD)
    ideg = deg[2 * _NPAD:].reshape(2, _NPAD)
    h1p = _mm_call(X, _oinv_call(odeg), W0)
    agg1 = _agg_hid(h1p, src, dst)
    h2p = _l2_call(agg1, odeg, ideg, b0, W1)
    agg2 = _agg_out(h2p, src, dst)
    return _fin_call(agg2, ideg, b1)


# speedup vs baseline: 16.3628x; 1.0146x over previous
import jax
import jax.numpy as jnp
from jax import lax
from jax.experimental import pallas as pl
from jax.experimental.pallas import tpu as pltpu
from jax.experimental.pallas import tpu_sc as plsc

_N = 10000
_E = 640000
_D_IN = 1433
_D_HID = 140
_D_OUT = 7

_NC = 2
_NS = 16
_NW = _NC * _NS
_NPAD = 10240
_RPT = _NPAD // _NS
_D1 = 144
_D2 = 16
_K = 80
_EPW = _E // _NW


def _deg_body(src_hbm, dst_hbm, deg_out, *scratch):
    G = 25
    idx_s = scratch[:G]
    idx_d = scratch[G:2 * G]
    ones_v, zbuf, sp_o, sp_i, isem, ssem = scratch[2 * G:]
    cid = lax.axis_index("c")
    sid = lax.axis_index("s")
    wid = cid * _NS + sid
    for i in range(_K // 16):
        ones_v[pl.ds(16 * i, 16)] = jnp.ones((16,), jnp.float32)
    for i in range(_RPT // 16):
        zbuf[pl.ds(16 * i, 16)] = jnp.zeros((16,), jnp.float32)
    pltpu.sync_copy(zbuf, sp_o.at[pl.ds(sid * _RPT, _RPT)])
    pltpu.sync_copy(zbuf, sp_i.at[pl.ds(sid * _RPT, _RPT)])
    plsc.subcore_barrier()

    def body(t, carry):
        base = wid * _EPW + t * (G * _K)
        ld = [pltpu.async_copy(src_hbm.at[pl.ds(base + g * _K, _K)],
                               idx_s[g], isem) for g in range(G)]
        ld += [pltpu.async_copy(dst_hbm.at[pl.ds(base + g * _K, _K)],
                                idx_d[g], isem) for g in range(G)]
        for d in ld:
            d.wait()
        sd = [pltpu.async_copy(ones_v, sp_o.at[idx_s[g]], ssem, add=True)
              for g in range(G)]
        sd += [pltpu.async_copy(ones_v, sp_i.at[idx_d[g]], ssem, add=True)
               for g in range(G)]
        for d in sd:
            d.wait()
        return carry

    lax.fori_loop(0, _EPW // (G * _K), body, 0)
    plsc.subcore_barrier()
    off = cid * _NPAD + sid * _RPT
    pltpu.sync_copy(sp_o.at[pl.ds(sid * _RPT, _RPT)],
                    deg_out.at[pl.ds(off, _RPT)])
    pltpu.sync_copy(sp_i.at[pl.ds(sid * _RPT, _RPT)],
                    deg_out.at[pl.ds(2 * _NPAD + off, _RPT)])


_SC_PARAMS = pltpu.CompilerParams(use_tc_tiling_on_sc=False)

_deg_call = pl.kernel(
    _deg_body,
    out_type=jax.ShapeDtypeStruct((4 * _NPAD,), jnp.float32),
    mesh=plsc.VectorSubcoreMesh(core_axis_name="c", subcore_axis_name="s"),
    compiler_params=_SC_PARAMS,
    scratch_types=(
        [pltpu.VMEM((_K,), jnp.int32) for _ in range(50)]
        + [
            pltpu.VMEM((_K,), jnp.float32),
            pltpu.VMEM((_RPT,), jnp.float32),
            pltpu.VMEM_SHARED((_NPAD,), jnp.float32),
            pltpu.VMEM_SHARED((_NPAD,), jnp.float32),
            pltpu.SemaphoreType.DMA,
            pltpu.SemaphoreType.DMA,
        ]
    ),
)


def _agg1_body(h_hbm, src_hbm, dst_hbm, agg_out, *scratch):
    D = _D1
    NCH = _EPW // _K
    idx_s = scratch[0:4]
    idx_d = scratch[4:8]
    stage = scratch[8:10]
    z8, sp_agg, isem, gsem, ssem = scratch[10:]
    cid = lax.axis_index("c")
    sid = lax.axis_index("s")
    wid = cid * _NS + sid
    base = wid * _EPW
    for i in range(8 * D // 16):
        z8[i // (D // 16), pl.ds(16 * (i % (D // 16)), 16)] = (
            jnp.zeros((16,), jnp.float32))
    zd = [pltpu.async_copy(z8, sp_agg.at[pl.ds(sid * _RPT + 8 * r, 8)],
                           gsem) for r in range(_RPT // 8)]
    for d in zd:
        d.wait()
    plsc.subcore_barrier()

    def fire_idx(c, slot):
        pltpu.async_copy(src_hbm.at[pl.ds(base + c * _K, _K)],
                         idx_s[slot], isem)
        pltpu.async_copy(dst_hbm.at[pl.ds(base + c * _K, _K)],
                         idx_d[slot], isem)

    def wait_idx(slot):
        pltpu.make_async_copy(src_hbm.at[pl.ds(base, _K)],
                              idx_s[slot], isem).wait()
        pltpu.make_async_copy(dst_hbm.at[pl.ds(base, _K)],
                              idx_d[slot], isem).wait()

    def wait_scatter(slot2, slot4):
        pltpu.make_async_copy(stage[slot2], sp_agg.at[idx_d[slot4]],
                              ssem).wait()

    for c in range(4):
        fire_idx(c, c)
    for c in range(2):
        wait_idx(c)
        pltpu.async_copy(h_hbm.at[idx_s[c]], stage[c], gsem).wait()
        pltpu.async_copy(stage[c], sp_agg.at[idx_d[c]], ssem, add=True)

    def body_t(t, carry):
        for j in range(4):
            sl4 = (j + 2) % 4
            sl2 = j % 2
            c = 4 * t + j + 2
            wait_idx(sl4)
            wait_scatter(sl2, j)
            g = pltpu.async_copy(h_hbm.at[idx_s[sl4]], stage[sl2], gsem)

            @pl.when(c + 2 <= NCH - 1)
            def _():
                fire_idx(c + 2, j)

            g.wait()
            pltpu.async_copy(stage[sl2], sp_agg.at[idx_d[sl4]], ssem,
                             add=True)
        return carry

    lax.fori_loop(0, (NCH - 2) // 4, body_t, 0)
    wait_scatter(0, 0)
    wait_scatter(1, 1)
    plsc.subcore_barrier()
    pltpu.sync_copy(sp_agg.at[pl.ds(sid * _RPT, _RPT)],
                    agg_out.at[cid, pl.ds(sid * _RPT, _RPT)])


_agg_hid = pl.kernel(
    _agg1_body,
    out_type=jax.ShapeDtypeStruct((_NC, _NPAD, _D1), jnp.float32),
    mesh=plsc.VectorSubcoreMesh(core_axis_name="c", subcore_axis_name="s"),
    compiler_params=_SC_PARAMS,
    scratch_types=(
        [pltpu.VMEM((_K,), jnp.int32) for _ in range(8)]
        + [pltpu.VMEM((_K, _D1), jnp.float32) for _ in range(2)]
        + [
            pltpu.VMEM((8, _D1), jnp.float32),
            pltpu.VMEM_SHARED((_NPAD, _D1), jnp.float32),
            pltpu.SemaphoreType.DMA,
            pltpu.SemaphoreType.DMA,
            pltpu.SemaphoreType.DMA,
        ]
    ),
)


def _make_agg(D, G):
    def body(h_hbm, src_hbm, dst_hbm, agg_out, *scratch):
        idx_s = scratch[:G]
        idx_d = scratch[G:2 * G]
        stage = scratch[2 * G:3 * G]
        z8, sp_agg, isem, gsem, ssem = scratch[3 * G:]
        cid = lax.axis_index("c")
        sid = lax.axis_index("s")
        wid = cid * _NS + sid
        for i in range(8 * D // 16):
            z8[i // (D // 16), pl.ds(16 * (i % (D // 16)), 16)] = (
                jnp.zeros((16,), jnp.float32))
        zd = [pltpu.async_copy(z8, sp_agg.at[pl.ds(sid * _RPT + 8 * r, 8)],
                               gsem) for r in range(_RPT // 8)]
        for d in zd:
            d.wait()
        plsc.subcore_barrier()

        def body_t(t, carry):
            base = wid * _EPW + t * (G * _K)
            ld = [pltpu.async_copy(src_hbm.at[pl.ds(base + g * _K, _K)],
                                   idx_s[g], isem) for g in range(G)]
            ld += [pltpu.async_copy(dst_hbm.at[pl.ds(base + g * _K, _K)],
                                    idx_d[g], isem) for g in range(G)]
            for d in ld:
                d.wait()
            gd = [pltpu.async_copy(h_hbm.at[idx_s[g]], stage[g], gsem)
                  for g in range(G)]
            for d in gd:
                d.wait()
            sd = [pltpu.async_copy(stage[g], sp_agg.at[idx_d[g]], ssem,
                                   add=True) for g in range(G)]
            for d in sd:
                d.wait()
            return carry

        lax.fori_loop(0, _EPW // (G * _K), body_t, 0)
        plsc.subcore_barrier()
        pltpu.sync_copy(sp_agg.at[pl.ds(sid * _RPT, _RPT)],
                        agg_out.at[cid, pl.ds(sid * _RPT, _RPT)])

    return pl.kernel(
        body,
        out_type=jax.ShapeDtypeStruct((_NC, _NPAD, D), jnp.float32),
        mesh=plsc.VectorSubcoreMesh(core_axis_name="c", subcore_axis_name="s"),
        compiler_params=_SC_PARAMS,
        scratch_types=(
            [pltpu.VMEM((_K,), jnp.int32) for _ in range(2 * G)]
            + [pltpu.VMEM((_K, D), jnp.float32) for _ in range(G)]
            + [
                pltpu.VMEM((8, D), jnp.float32),
                pltpu.VMEM_SHARED((_NPAD, D), jnp.float32),
                pltpu.SemaphoreType.DMA,
                pltpu.SemaphoreType.DMA,
                pltpu.SemaphoreType.DMA,
            ]
        ),
    )


_agg_out = _make_agg(_D2, 50)


def _mm_body(x_ref, od_ref, w_ref, o_ref, s_ref):
    i = pl.program_id(0)

    @pl.when(i == 0)
    def _():
        od = od_ref[0, :] + od_ref[1, :]
        s_ref[...] = lax.rsqrt(jnp.maximum(od, 1.0))[:, None]

    xs = x_ref[...] * s_ref[pl.ds(i * 1000, 1000), :]
    o_ref[:, :_D_HID] = jnp.dot(xs, w_ref[...],
                                preferred_element_type=jnp.float32)
    o_ref[:, _D_HID:] = jnp.zeros((1000, _D1 - _D_HID), jnp.float32)


def _mm_call(X, odeg, W0):
    return pl.pallas_call(
        _mm_body,
        grid=(10,),
        in_specs=[pl.BlockSpec((1000, _D_IN), lambda i: (i, 0)),
                  pl.BlockSpec((2, _NPAD), lambda i: (0, 0)),
                  pl.BlockSpec((_D_IN, _D_HID), lambda i: (0, 0))],
        out_specs=pl.BlockSpec((1000, _D1), lambda i: (i, 0)),
        out_shape=jax.ShapeDtypeStruct((_N, _D1), jnp.float32),
        scratch_shapes=[pltpu.VMEM((_NPAD, 1), jnp.float32)],
    )(X, odeg, W0)


def _l2_body(agg_ref, od_ref, id_ref, b0_ref, w1_ref, h2_ref):
    a = agg_ref[0, :_N, :_D_HID] + agg_ref[1, :_N, :_D_HID]
    iinv = lax.rsqrt(jnp.maximum(id_ref[0, :] + id_ref[1, :], 1.0))[:_N, None]
    oinv = lax.rsqrt(jnp.maximum(od_ref[0, :] + od_ref[1, :], 1.0))[:_N, None]
    h1 = jnp.maximum(a * iinv + b0_ref[...][None, :], 0.0)
    m2 = jnp.dot(h1 * oinv, w1_ref[...], preferred_element_type=jnp.float32)
    h2_ref[:, :_D_OUT] = m2
    h2_ref[:, _D_OUT:] = jnp.zeros((_N, _D2 - _D_OUT), jnp.float32)


def _l2_call(agg1, odeg, ideg, b0, W1):
    return pl.pallas_call(
        _l2_body,
        out_shape=jax.ShapeDtypeStruct((_N, _D2), jnp.float32),
    )(agg1, odeg, ideg, b0, W1)


def _fin_body(agg_ref, id_ref, b1_ref, o_ref):
    a = agg_ref[0, :_N, :_D_OUT] + agg_ref[1, :_N, :_D_OUT]
    iinv = lax.rsqrt(jnp.maximum(id_ref[0, :] + id_ref[1, :], 1.0))[:_N, None]
    o_ref[...] = jnp.maximum(a * iinv + b1_ref[...][None, :], 0.0)


def _fin_call(agg2, ideg, b1):
    return pl.pallas_call(
        _fin_body,
        out_shape=jax.ShapeDtypeStruct((_N, _D_OUT), jnp.float32),
    )(agg2, ideg, b1)


def kernel(X, edge_index, W0, b0, W1, b1):
    src = edge_index[0]
    dst = edge_index[1]
    deg = _deg_call(src, dst)
    odeg = deg[:2 * _NPAD].reshape(2, _NPAD)
    ideg = deg[2 * _NPAD:].reshape(2, _NPAD)
    h1p = _mm_call(X, odeg, W0)
    agg1 = _agg_hid(h1p, src, dst)
    h2p = _l2_call(agg1, odeg, ideg, b0, W1)
    agg2 = _agg_out(h2p, src, dst)
    return _fin_call(agg2, ideg, b1)
